# Initial kernel scaffold; baseline (speedup 1.0000x reference)
import jax
import jax.numpy as jnp
from jax.experimental import pallas as pl


def kernel(x, wave_w, wave_b, kernel_w, kernel_b, out_w):
    raise NotImplementedError("write your pallas kernel here")


# speedup vs baseline: 4.9230x; 4.9230x over previous
import functools

import jax
import jax.numpy as jnp
from jax import lax
from jax.experimental import pallas as pl
from jax.experimental.pallas import tpu as pltpu
from jax.experimental.pallas import tpu_sc as plsc

L = 2048
C = 1024
H = 8
D = C // H
K = 64
HALF_S = 16
S = 2 * HALF_S + 1
MAX_FREQ = 16.0
MIN_FREQ = 1.0
MAX_RECEPTIVE = HALF_S * MAX_FREQ

TM = 256
TM3 = 512
W72 = 72
ITEMS = L * H
NW = 32
IPW = ITEMS // NW
NB = 4
NSTEP = IPW // NB


def _silu(v):
    return v * jax.nn.sigmoid(v)


def _prep_body(x_ref, ww_ref, wb_ref, kw_ref, kb_ref,
               i0_ref, i1_ref, w0_ref, w1_ref):
    t0 = pl.program_id(0) * TM
    xt = x_ref[...]
    wave = lax.dot_general(xt, ww_ref[...], (((1,), (1,)), ((), ())),
                           preferred_element_type=jnp.float32) + wb_ref[...]
    wave = _silu(wave)
    freq = jax.nn.sigmoid(wave[:, 0:H]) * (MAX_FREQ - MIN_FREQ) + MIN_FREQ
    phase = jnp.tanh(wave[:, H:2 * H]) * MAX_FREQ
    kern = lax.dot_general(xt, kw_ref[...], (((1,), (1,)), ((), ())),
                           preferred_element_type=jnp.float32) + kb_ref[...]
    ktap = _silu(kern).reshape(TM, H, S)

    sv = lax.broadcasted_iota(jnp.int32, (TM, H, S), 2).astype(jnp.float32) - float(HALF_S)
    off = sv * freq[:, :, None] + phase[:, :, None]
    off = jnp.clip(off, -MAX_RECEPTIVE, MAX_RECEPTIVE)
    pos = (lax.broadcasted_iota(jnp.int32, (TM, H, S), 0) + t0).astype(jnp.float32)
    p = jnp.clip(pos + off, 0.0, float(L - 1))
    i0f = jnp.floor(p)
    frac = p - i0f
    i0 = i0f.astype(jnp.int32)
    i1 = jnp.minimum(i0 + 1, L - 1)
    hof = lax.broadcasted_iota(jnp.int32, (TM, H, S), 1) * L
    i0_ref[...] = i0 + hof
    i1_ref[...] = i1 + hof
    w0_ref[...] = ktap * (1.0 - frac)
    w1_ref[...] = ktap * frac


def _prep(xt, ww, wb, kw, kb):
    grid = L // TM
    f32 = jnp.float32
    return pl.pallas_call(
        _prep_body,
        grid=(grid,),
        in_specs=[
            pl.BlockSpec((TM, C), lambda i: (i, 0)),
            pl.BlockSpec((2 * H, C), lambda i: (0, 0)),
            pl.BlockSpec((1, 2 * H), lambda i: (0, 0)),
            pl.BlockSpec((H * S, C), lambda i: (0, 0)),
            pl.BlockSpec((1, H * S), lambda i: (0, 0)),
        ],
        out_specs=[
            pl.BlockSpec((TM, H, S), lambda i: (i, 0, 0)),
            pl.BlockSpec((TM, H, S), lambda i: (i, 0, 0)),
            pl.BlockSpec((TM, H, S), lambda i: (i, 0, 0)),
            pl.BlockSpec((TM, H, S), lambda i: (i, 0, 0)),
        ],
        out_shape=[
            jax.ShapeDtypeStruct((L, H, S), jnp.int32),
            jax.ShapeDtypeStruct((L, H, S), jnp.int32),
            jax.ShapeDtypeStruct((L, H, S), f32),
            jax.ShapeDtypeStruct((L, H, S), f32),
        ],
    )(xt, ww, wb, kw, kb)


_sc_mesh = plsc.VectorSubcoreMesh(core_axis_name="c", subcore_axis_name="s")


@functools.partial(
    pl.kernel,
    out_type=jax.ShapeDtypeStruct((ITEMS, D), jnp.float32),
    mesh=_sc_mesh,
    scratch_types=[
        pltpu.VMEM((NB * W72,), jnp.int32),
        pltpu.VMEM((NB * W72 + 16,), jnp.float32),
        pltpu.VMEM((NB * W72, D), jnp.float32),
        pltpu.VMEM((NB, D), jnp.float32),
        pltpu.SemaphoreType.DMA,
    ],
)
def _sc_gather(table, idxs, ws, out, idx_v, w_v, rows_v, acc_v, sem):
    wid = lax.axis_index("s") * 2 + lax.axis_index("c")
    base = wid * IPW

    def step(st, carry):
        it = base + st * NB
        pltpu.sync_copy(idxs.at[pl.ds(it * W72, NB * W72)], idx_v)
        pltpu.sync_copy(ws.at[pl.ds(it * W72, NB * W72)], w_v.at[pl.ds(0, NB * W72)])
        pltpu.async_copy(table.at[idx_v], rows_v, sem).wait()
        for j in range(NB):
            def sbody(s, acc, j=j):
                w0 = w_v[pl.ds(j * W72 + s, 16)][0]
                w1 = w_v[pl.ds(j * W72 + s + S, 16)][0]
                return tuple(
                    acc[q]
                    + w0 * rows_v[j * W72 + s, pl.ds(q * 16, 16)]
                    + w1 * rows_v[j * W72 + s + S, pl.ds(q * 16, 16)]
                    for q in range(8))
            acc = lax.fori_loop(
                0, S, sbody,
                tuple(jnp.zeros((16,), jnp.float32) for _ in range(8)))
            for q in range(8):
                acc_v[j, pl.ds(q * 16, 16)] = acc[q]
        pltpu.sync_copy(acc_v, out.at[pl.ds(it, NB)])
        return carry

    lax.fori_loop(0, NSTEP, step, 0)


def _out_body(h_ref, w_ref, o_ref):
    acc = lax.dot_general(h_ref[...], w_ref[...], (((1,), (1,)), ((), ())),
                          preferred_element_type=jnp.float32)
    o_ref[...] = _silu(acc)


def _outproj(hid2, ow):
    grid = L // TM3
    return pl.pallas_call(
        _out_body,
        grid=(grid,),
        in_specs=[
            pl.BlockSpec((TM3, C), lambda i: (i, 0)),
            pl.BlockSpec((C, C), lambda i: (0, 0)),
        ],
        out_specs=pl.BlockSpec((TM3, C), lambda i: (i, 0)),
        out_shape=jax.ShapeDtypeStruct((L, C), jnp.float32),
    )(hid2, ow)


def kernel(x, wave_w, wave_b, kernel_w, kernel_b, out_w):
    xt = x[0]
    kw = kernel_w.reshape(H, K, C)[:, :S].reshape(H * S, C)
    kb = kernel_b.reshape(H, K)[:, :S].reshape(1, H * S)
    wb = wave_b.reshape(1, 2 * H)

    i0g, i1g, w0, w1 = _prep(xt, wave_w, wb, kw, kb)

    pad_i = jnp.zeros((L, H, W72 - 2 * S), jnp.int32)
    pad_f = jnp.zeros((L, H, W72 - 2 * S), jnp.float32)
    idx_all = jnp.concatenate([i0g, i1g, pad_i], axis=-1).reshape(ITEMS * W72)
    w_all = jnp.concatenate([w0, w1, pad_f], axis=-1).reshape(ITEMS * W72)
    table = xt.reshape(L, H, D).transpose(1, 0, 2).reshape(H * L, D)

    hid = _sc_gather(table, idx_all, w_all)
    out = _outproj(hid.reshape(L, C), out_w)
    return out[None]


# speedup vs baseline: 56.4217x; 11.4608x over previous
import functools

import jax
import jax.numpy as jnp
from jax import lax
from jax.experimental import pallas as pl
from jax.experimental.pallas import tpu as pltpu
from jax.experimental.pallas import tpu_sc as plsc

L = 2048
C = 1024
H = 8
D = C // H
K = 64
HALF_S = 16
S = 2 * HALF_S + 1
MAX_FREQ = 16.0
MIN_FREQ = 1.0
MAX_RECEPTIVE = HALF_S * MAX_FREQ

TM = 256
TM3 = 512
SP = 40
ITEMS = L * H
NW = 32
TPW = L // 4
TB = 128
NBLK = TPW // TB
WIN = 656
WS_OFF = 264


def _silu(v):
    return v * jax.nn.sigmoid(v)


def _prep_body(x_ref, ww_ref, wb_ref, kw_ref, kb_ref,
               r0_ref, w0_ref, w1_ref):
    t0 = pl.program_id(0) * TM
    xt = x_ref[...]
    wave = lax.dot_general(xt, ww_ref[...], (((1,), (1,)), ((), ())),
                           preferred_element_type=jnp.float32) + wb_ref[...]
    wave = _silu(wave)
    freq = jax.nn.sigmoid(wave[:, 0:H]) * (MAX_FREQ - MIN_FREQ) + MIN_FREQ
    phase = jnp.tanh(wave[:, H:2 * H]) * MAX_FREQ
    kern = lax.dot_general(xt, kw_ref[...], (((1,), (1,)), ((), ())),
                           preferred_element_type=jnp.float32) + kb_ref[...]
    ktap = _silu(kern).reshape(TM, H, S)

    sv = lax.broadcasted_iota(jnp.int32, (TM, H, S), 2).astype(jnp.float32) - float(HALF_S)
    off = sv * freq[:, :, None] + phase[:, :, None]
    off = jnp.clip(off, -MAX_RECEPTIVE, MAX_RECEPTIVE)
    ti = lax.broadcasted_iota(jnp.int32, (TM, H, S), 0) + t0
    p = jnp.clip(ti.astype(jnp.float32) + off, 0.0, float(L - 1))
    i0f = jnp.floor(p)
    frac = p - i0f
    i0 = i0f.astype(jnp.int32)
    ws = jnp.clip((ti // TB) * TB - WS_OFF, 0, L - WIN)
    r0_ref[...] = i0 - ws
    w0_ref[...] = ktap * (1.0 - frac)
    w1_ref[...] = ktap * frac


def _prep(xt, ww, wb, kw, kb):
    grid = L // TM
    return pl.pallas_call(
        _prep_body,
        grid=(grid,),
        in_specs=[
            pl.BlockSpec((TM, C), lambda i: (i, 0)),
            pl.BlockSpec((2 * H, C), lambda i: (0, 0)),
            pl.BlockSpec((1, 2 * H), lambda i: (0, 0)),
            pl.BlockSpec((H * S, C), lambda i: (0, 0)),
            pl.BlockSpec((1, H * S), lambda i: (0, 0)),
        ],
        out_specs=[
            pl.BlockSpec((TM, H, S), lambda i: (i, 0, 0)),
            pl.BlockSpec((TM, H, S), lambda i: (i, 0, 0)),
            pl.BlockSpec((TM, H, S), lambda i: (i, 0, 0)),
        ],
        out_shape=[
            jax.ShapeDtypeStruct((L, H, S), jnp.int32),
            jax.ShapeDtypeStruct((L, H, S), jnp.float32),
            jax.ShapeDtypeStruct((L, H, S), jnp.float32),
        ],
    )(xt, ww, wb, kw, kb)


_sc_mesh = plsc.VectorSubcoreMesh(core_axis_name="c", subcore_axis_name="s")


@functools.partial(
    pl.kernel,
    out_type=jax.ShapeDtypeStruct((ITEMS, D), jnp.float32),
    mesh=_sc_mesh,
    scratch_types=[
        pltpu.VMEM((WIN, D), jnp.float32),
        pltpu.VMEM((TB * SP + 16,), jnp.int32),
        pltpu.VMEM((TB * SP + 16,), jnp.float32),
        pltpu.VMEM((TB * SP + 16,), jnp.float32),
        pltpu.VMEM((TB, D), jnp.float32),
        pltpu.SemaphoreType.DMA,
    ],
)
def _sc_gather(table, r0s, w0s, w1s, out, win_v, r_v, w0_v, w1_v, out_v, sem):
    wid = lax.axis_index("s") * 2 + lax.axis_index("c")
    h = wid // 4
    tw = (wid % 4) * TPW

    def blk(b, carry):
        tb = tw + b * TB
        ws = jnp.clip(tb - WS_OFF, 0, L - WIN)
        pltpu.sync_copy(table.at[pl.ds(pl.multiple_of(h * L + ws, 8), WIN)], win_v)
        base = pl.multiple_of((h * L + tb) * SP, 8)
        pltpu.sync_copy(r0s.at[pl.ds(base, TB * SP)], r_v.at[pl.ds(0, TB * SP)])
        pltpu.sync_copy(w0s.at[pl.ds(base, TB * SP)], w0_v.at[pl.ds(0, TB * SP)])
        pltpu.sync_copy(w1s.at[pl.ds(base, TB * SP)], w1_v.at[pl.ds(0, TB * SP)])

        def tok(tl, carry2):
            toff = tl * SP

            def sbody(s, acc):
                r0 = r_v[pl.ds(toff + s, 16)][0]
                w0 = w0_v[pl.ds(toff + s, 16)][0]
                w1 = w1_v[pl.ds(toff + s, 16)][0]
                return tuple(
                    acc[q]
                    + w0 * win_v[r0, pl.ds(q * 16, 16)]
                    + w1 * win_v[r0 + 1, pl.ds(q * 16, 16)]
                    for q in range(8))

            acc = lax.fori_loop(
                0, S, sbody,
                tuple(jnp.zeros((16,), jnp.float32) for _ in range(8)))
            for q in range(8):
                out_v[tl, pl.ds(q * 16, 16)] = acc[q]
            return carry2

        lax.fori_loop(0, TB, tok, 0)
        pltpu.sync_copy(out_v, out.at[pl.ds(pl.multiple_of(h * L + tb, 8), TB)])
        return carry

    lax.fori_loop(0, NBLK, blk, 0)


def _out_body(h_ref, w_ref, o_ref):
    acc = lax.dot_general(h_ref[...], w_ref[...], (((1,), (1,)), ((), ())),
                          preferred_element_type=jnp.float32)
    o_ref[...] = _silu(acc)


def _outproj(hid2, ow):
    grid = L // TM3
    return pl.pallas_call(
        _out_body,
        grid=(grid,),
        in_specs=[
            pl.BlockSpec((TM3, C), lambda i: (i, 0)),
            pl.BlockSpec((C, C), lambda i: (0, 0)),
        ],
        out_specs=pl.BlockSpec((TM3, C), lambda i: (i, 0)),
        out_shape=jax.ShapeDtypeStruct((L, C), jnp.float32),
    )(hid2, ow)


def kernel(x, wave_w, wave_b, kernel_w, kernel_b, out_w):
    xt = x[0]
    kw = kernel_w.reshape(H, K, C)[:, :S].reshape(H * S, C)
    kb = kernel_b.reshape(H, K)[:, :S].reshape(1, H * S)
    wb = wave_b.reshape(1, 2 * H)

    r0, w0, w1 = _prep(xt, wave_w, wb, kw, kb)

    pad = ((0, 0), (0, 0), (0, SP - S))
    r0p = jnp.pad(r0, pad).transpose(1, 0, 2).reshape(H * L * SP)
    w0p = jnp.pad(w0, pad).transpose(1, 0, 2).reshape(H * L * SP)
    w1p = jnp.pad(w1, pad).transpose(1, 0, 2).reshape(H * L * SP)
    table = xt.reshape(L, H, D).transpose(1, 0, 2).reshape(H * L, D)

    hid = _sc_gather(table, r0p, w0p, w1p)
    hid2 = hid.reshape(H, L, D).transpose(1, 0, 2).reshape(L, C)
    out = _outproj(hid2, out_w)
    return out[None]


# speedup vs baseline: 57.3322x; 1.0161x over previous
import functools

import jax
import jax.numpy as jnp
from jax import lax
from jax.experimental import pallas as pl
from jax.experimental.pallas import tpu as pltpu
from jax.experimental.pallas import tpu_sc as plsc

L = 2048
C = 1024
H = 8
D = C // H
K = 64
HALF_S = 16
S = 2 * HALF_S + 1
MAX_FREQ = 16.0
MIN_FREQ = 1.0
MAX_RECEPTIVE = HALF_S * MAX_FREQ

TM = 256
TM3 = 512
SP = 40
ITEMS = L * H
NW = 32
TPW = L // 4
TB = 128
NBLK = TPW // TB
WIN = 656
WS_OFF = 264


def _silu(v):
    return v * jax.nn.sigmoid(v)


def _prep_body(x_ref, ww_ref, wb_ref, kw_ref, kb_ref,
               r0_ref, w0_ref, w1_ref):
    t0 = pl.program_id(0) * TM
    j = pl.program_id(1)
    xt = x_ref[...]
    wave = lax.dot_general(xt, ww_ref[...], (((1,), (1,)), ((), ())),
                           preferred_element_type=jnp.float32) + wb_ref[...]
    wave = _silu(wave)
    freq = jax.nn.sigmoid(wave[:, 0:H]) * (MAX_FREQ - MIN_FREQ) + MIN_FREQ
    phase = jnp.tanh(wave[:, H:2 * H]) * MAX_FREQ
    kern = lax.dot_general(xt, kw_ref[0], (((1,), (1,)), ((), ())),
                           preferred_element_type=jnp.float32) + kb_ref[0]
    ktap = _silu(kern)

    oh = (lax.broadcasted_iota(jnp.int32, (H, S), 0) == j).astype(jnp.float32)
    freq_s = lax.dot_general(freq, oh, (((1,), (0,)), ((), ())),
                             precision=lax.Precision.HIGHEST,
                             preferred_element_type=jnp.float32)
    phase_s = lax.dot_general(phase, oh, (((1,), (0,)), ((), ())),
                              precision=lax.Precision.HIGHEST,
                              preferred_element_type=jnp.float32)
    sv = lax.broadcasted_iota(jnp.int32, (TM, S), 1).astype(jnp.float32) \
        - float(HALF_S)
    off = sv * freq_s + phase_s
    off = jnp.clip(off, -MAX_RECEPTIVE, MAX_RECEPTIVE)
    ti = lax.broadcasted_iota(jnp.int32, (TM, S), 0) + t0
    p = jnp.clip(ti.astype(jnp.float32) + off, 0.0, float(L - 1))
    i0f = jnp.floor(p)
    frac = p - i0f
    i0 = i0f.astype(jnp.int32)
    ws = jnp.clip((ti // TB) * TB - WS_OFF, 0, L - WIN)
    zi = jnp.zeros((TM, SP - S), jnp.int32)
    zf = jnp.zeros((TM, SP - S), jnp.float32)
    r0_ref[...] = jnp.concatenate([i0 - ws, zi], axis=1).reshape(1, TM, SP)
    w0_ref[...] = jnp.concatenate([ktap * (1.0 - frac), zf], axis=1).reshape(1, TM, SP)
    w1_ref[...] = jnp.concatenate([ktap * frac, zf], axis=1).reshape(1, TM, SP)


def _prep(xt, ww, wb, kw, kb):
    return pl.pallas_call(
        _prep_body,
        grid=(L // TM, H),
        in_specs=[
            pl.BlockSpec((TM, C), lambda i, j: (i, 0)),
            pl.BlockSpec((2 * H, C), lambda i, j: (0, 0)),
            pl.BlockSpec((1, 2 * H), lambda i, j: (0, 0)),
            pl.BlockSpec((1, S, C), lambda i, j: (j, 0, 0)),
            pl.BlockSpec((1, 1, S), lambda i, j: (j, 0, 0)),
        ],
        out_specs=[
            pl.BlockSpec((1, TM, SP), lambda i, j: (j, i, 0)),
            pl.BlockSpec((1, TM, SP), lambda i, j: (j, i, 0)),
            pl.BlockSpec((1, TM, SP), lambda i, j: (j, i, 0)),
        ],
        out_shape=[
            jax.ShapeDtypeStruct((H, L, SP), jnp.int32),
            jax.ShapeDtypeStruct((H, L, SP), jnp.float32),
            jax.ShapeDtypeStruct((H, L, SP), jnp.float32),
        ],
    )(xt, ww, wb, kw.reshape(H, S, C), kb.reshape(H, 1, S))


_sc_mesh = plsc.VectorSubcoreMesh(core_axis_name="c", subcore_axis_name="s")


@functools.partial(
    pl.kernel,
    out_type=jax.ShapeDtypeStruct((ITEMS, D), jnp.float32),
    mesh=_sc_mesh,
    scratch_types=[
        pltpu.VMEM((WIN, D), jnp.float32),
        pltpu.VMEM((TB * SP + 16,), jnp.int32),
        pltpu.VMEM((TB * SP + 16,), jnp.float32),
        pltpu.VMEM((TB * SP + 16,), jnp.float32),
        pltpu.VMEM((TB, D), jnp.float32),
        pltpu.SemaphoreType.DMA,
    ],
)
def _sc_gather(table, r0s, w0s, w1s, out, win_v, r_v, w0_v, w1_v, out_v, sem):
    wid = lax.axis_index("s") * 2 + lax.axis_index("c")
    h = wid // 4
    tw = (wid % 4) * TPW

    def blk(b, carry):
        tb = tw + b * TB
        ws = jnp.clip(tb - WS_OFF, 0, L - WIN)
        pltpu.sync_copy(
            table.at[pl.ds(pl.multiple_of(ws, 8), WIN),
                     pl.ds(pl.multiple_of(h * D, 128), D)],
            win_v)
        base = pl.multiple_of((h * L + tb) * SP, 8)
        pltpu.sync_copy(r0s.at[pl.ds(base, TB * SP)], r_v.at[pl.ds(0, TB * SP)])
        pltpu.sync_copy(w0s.at[pl.ds(base, TB * SP)], w0_v.at[pl.ds(0, TB * SP)])
        pltpu.sync_copy(w1s.at[pl.ds(base, TB * SP)], w1_v.at[pl.ds(0, TB * SP)])

        def tok(tl, carry2):
            toff = tl * SP

            def sbody(s, acc):
                r0 = r_v[pl.ds(toff + s, 16)][0]
                w0 = w0_v[pl.ds(toff + s, 16)][0]
                w1 = w1_v[pl.ds(toff + s, 16)][0]
                return tuple(
                    acc[q]
                    + w0 * win_v[r0, pl.ds(q * 16, 16)]
                    + w1 * win_v[r0 + 1, pl.ds(q * 16, 16)]
                    for q in range(8))

            acc = lax.fori_loop(
                0, S, sbody,
                tuple(jnp.zeros((16,), jnp.float32) for _ in range(8)))
            for q in range(8):
                out_v[tl, pl.ds(q * 16, 16)] = acc[q]
            return carry2

        lax.fori_loop(0, TB, tok, 0)
        pltpu.sync_copy(out_v, out.at[pl.ds(pl.multiple_of(h * L + tb, 8), TB)])
        return carry

    lax.fori_loop(0, NBLK, blk, 0)


def _out_body(h_ref, w_ref, o_ref):
    acc = lax.dot_general(h_ref[0], w_ref[:, 0:D],
                          (((1,), (1,)), ((), ())),
                          preferred_element_type=jnp.float32)
    for hh in range(1, H):
        acc += lax.dot_general(h_ref[hh], w_ref[:, hh * D:(hh + 1) * D],
                               (((1,), (1,)), ((), ())),
                               preferred_element_type=jnp.float32)
    o_ref[...] = _silu(acc)


def _outproj(hid3, ow):
    return pl.pallas_call(
        _out_body,
        grid=(L // TM3,),
        in_specs=[
            pl.BlockSpec((H, TM3, D), lambda i: (0, i, 0)),
            pl.BlockSpec((C, C), lambda i: (0, 0)),
        ],
        out_specs=pl.BlockSpec((TM3, C), lambda i: (i, 0)),
        out_shape=jax.ShapeDtypeStruct((L, C), jnp.float32),
    )(hid3, ow)


def kernel(x, wave_w, wave_b, kernel_w, kernel_b, out_w):
    xt = x[0]
    kw = kernel_w.reshape(H, K, C)[:, :S].reshape(H * S, C)
    kb = kernel_b.reshape(H, K)[:, :S].reshape(1, H * S)
    wb = wave_b.reshape(1, 2 * H)

    r0p, w0p, w1p = _prep(xt, wave_w, wb, kw, kb)

    hid = _sc_gather(xt,
                     r0p.reshape(H * L * SP),
                     w0p.reshape(H * L * SP),
                     w1p.reshape(H * L * SP))
    out = _outproj(hid.reshape(H, L, D), out_w)
    return out[None]


# speedup vs baseline: 67.2668x; 1.1733x over previous
import functools

import jax
import jax.numpy as jnp
import numpy as np
from jax import lax
from jax.experimental import pallas as pl
from jax.experimental.pallas import tpu as pltpu
from jax.experimental.pallas import tpu_sc as plsc

L = 2048
C = 1024
H = 8
D = C // H
K = 64
HALF_S = 16
S = 2 * HALF_S + 1
MAX_FREQ = 16.0
MIN_FREQ = 1.0
MAX_RECEPTIVE = HALF_S * MAX_FREQ

TM = 256
TM3 = 512
SP = 40
ITEMS = L * H
NW = 32
TPW = L // 4
TB = 128
NBLK = TPW // TB
WIN = 656
WS_OFF = 264


def _silu(v):
    return v * jax.nn.sigmoid(v)


def _prep_body(x_ref, ww_ref, wb_ref, kw_ref, kb_ref,
               r0_ref, w0_ref, w1_ref):
    t0 = pl.program_id(0) * TM
    xt = x_ref[...]
    wave = lax.dot_general(xt, ww_ref[...], (((1,), (1,)), ((), ())),
                           preferred_element_type=jnp.float32) + wb_ref[...]
    wave = _silu(wave)
    freq = jax.nn.sigmoid(wave[:, 0:H]) * (MAX_FREQ - MIN_FREQ) + MIN_FREQ
    phase = jnp.tanh(wave[:, H:2 * H]) * MAX_FREQ
    kern = lax.dot_general(xt, kw_ref[...], (((1,), (1,)), ((), ())),
                           preferred_element_type=jnp.float32) + kb_ref[...]
    ktap_all = _silu(kern)

    sv = lax.broadcasted_iota(jnp.int32, (TM, S), 1).astype(jnp.float32) \
        - float(HALF_S)
    ti = lax.broadcasted_iota(jnp.int32, (TM, S), 0) + t0
    tif = ti.astype(jnp.float32)
    ws = jnp.clip((ti // TB) * TB - WS_OFF, 0, L - WIN)
    zi = jnp.zeros((TM, SP - S), jnp.int32)
    zf = jnp.zeros((TM, SP - S), jnp.float32)
    for hh in range(H):
        oh = (lax.broadcasted_iota(jnp.int32, (H, S), 0) == hh
              ).astype(jnp.float32)
        freq_s = lax.dot_general(freq, oh, (((1,), (0,)), ((), ())),
                                 precision=lax.Precision.HIGHEST,
                                 preferred_element_type=jnp.float32)
        phase_s = lax.dot_general(phase, oh, (((1,), (0,)), ((), ())),
                                  precision=lax.Precision.HIGHEST,
                                  preferred_element_type=jnp.float32)
        ktap = ktap_all[:, hh * S:(hh + 1) * S]
        off = jnp.clip(sv * freq_s + phase_s, -MAX_RECEPTIVE, MAX_RECEPTIVE)
        p = jnp.clip(tif + off, 0.0, float(L - 1))
        i0f = jnp.floor(p)
        frac = p - i0f
        i0 = i0f.astype(jnp.int32)
        r0_ref[hh] = jnp.concatenate([i0 - ws, zi], axis=1)
        w0_ref[hh] = jnp.concatenate([ktap * (1.0 - frac), zf], axis=1)
        w1_ref[hh] = jnp.concatenate([ktap * frac, zf], axis=1)


def _prep(xt, ww, wb, kw, kb):
    return pl.pallas_call(
        _prep_body,
        grid=(L // TM,),
        in_specs=[
            pl.BlockSpec((TM, C), lambda i: (i, 0)),
            pl.BlockSpec((2 * H, C), lambda i: (0, 0)),
            pl.BlockSpec((1, 2 * H), lambda i: (0, 0)),
            pl.BlockSpec((H * S, C), lambda i: (0, 0)),
            pl.BlockSpec((1, H * S), lambda i: (0, 0)),
        ],
        out_specs=[
            pl.BlockSpec((H, TM, SP), lambda i: (0, i, 0)),
            pl.BlockSpec((H, TM, SP), lambda i: (0, i, 0)),
            pl.BlockSpec((H, TM, SP), lambda i: (0, i, 0)),
        ],
        out_shape=[
            jax.ShapeDtypeStruct((H, L, SP), jnp.int32),
            jax.ShapeDtypeStruct((H, L, SP), jnp.float32),
            jax.ShapeDtypeStruct((H, L, SP), jnp.float32),
        ],
    )(xt, ww, wb, kw, kb)


_sc_mesh = plsc.VectorSubcoreMesh(core_axis_name="c", subcore_axis_name="s")


@functools.partial(
    pl.kernel,
    out_type=jax.ShapeDtypeStruct((ITEMS, D), jnp.float32),
    mesh=_sc_mesh,
    scratch_types=[
        pltpu.VMEM((WIN, D), jnp.float32),
        pltpu.VMEM((TB * SP + 16,), jnp.int32),
        pltpu.VMEM((TB * SP + 16,), jnp.float32),
        pltpu.VMEM((TB * SP + 16,), jnp.float32),
        pltpu.VMEM((TB, D), jnp.float32),
        pltpu.SemaphoreType.DMA,
    ],
)
def _sc_gather(table, r0s, w0s, w1s, out, win_v, r_v, w0_v, w1_v, out_v, sem):
    wid = lax.axis_index("s") * 2 + lax.axis_index("c")
    h = wid // 4
    tw = (wid % 4) * TPW

    def blk(b, carry):
        tb = tw + b * TB
        ws = jnp.clip(tb - WS_OFF, 0, L - WIN)
        base = pl.multiple_of((h * L + tb) * SP, 8)
        cps = [
            pltpu.async_copy(
                table.at[pl.ds(pl.multiple_of(ws, 8), WIN),
                         pl.ds(pl.multiple_of(h * D, 128), D)],
                win_v, sem),
            pltpu.async_copy(r0s.at[pl.ds(base, TB * SP)],
                             r_v.at[pl.ds(0, TB * SP)], sem),
            pltpu.async_copy(w0s.at[pl.ds(base, TB * SP)],
                             w0_v.at[pl.ds(0, TB * SP)], sem),
            pltpu.async_copy(w1s.at[pl.ds(base, TB * SP)],
                             w1_v.at[pl.ds(0, TB * SP)], sem),
        ]
        for cp in cps:
            cp.wait()

        def tok(tl, carry2):
            toff = tl * SP

            def sbody(s, acc):
                r0 = r_v[pl.ds(toff + s, 16)][0]
                w0 = w0_v[pl.ds(toff + s, 16)][0]
                w1 = w1_v[pl.ds(toff + s, 16)][0]
                return tuple(
                    acc[q]
                    + w0 * win_v[r0, pl.ds(q * 16, 16)]
                    + w1 * win_v[r0 + 1, pl.ds(q * 16, 16)]
                    for q in range(8))

            acc = lax.fori_loop(
                0, S, sbody,
                tuple(jnp.zeros((16,), jnp.float32) for _ in range(8)))
            for q in range(8):
                out_v[tl, pl.ds(q * 16, 16)] = acc[q]
            return carry2

        lax.fori_loop(0, TB, tok, 0)
        pltpu.sync_copy(out_v, out.at[pl.ds(pl.multiple_of(h * L + tb, 8), TB)])
        return carry

    lax.fori_loop(0, NBLK, blk, 0)


def _out_body(h_ref, w_ref, o_ref):
    acc = lax.dot_general(h_ref[0], w_ref[:, 0:D],
                          (((1,), (1,)), ((), ())),
                          preferred_element_type=jnp.float32)
    for hh in range(1, H):
        acc += lax.dot_general(h_ref[hh], w_ref[:, hh * D:(hh + 1) * D],
                               (((1,), (1,)), ((), ())),
                               preferred_element_type=jnp.float32)
    o_ref[...] = _silu(acc)


def _outproj(hid3, ow):
    return pl.pallas_call(
        _out_body,
        grid=(L // TM3,),
        in_specs=[
            pl.BlockSpec((H, TM3, D), lambda i: (0, i, 0)),
            pl.BlockSpec((C, C), lambda i: (0, 0)),
        ],
        out_specs=pl.BlockSpec((TM3, C), lambda i: (i, 0)),
        out_shape=jax.ShapeDtypeStruct((L, C), jnp.float32),
    )(hid3, ow)


def kernel(x, wave_w, wave_b, kernel_w, kernel_b, out_w):
    xt = x[0]
    kw = kernel_w.reshape(H, K, C)[:, :S].reshape(H * S, C)
    kb = kernel_b.reshape(H, K)[:, :S].reshape(1, H * S)
    wb = wave_b.reshape(1, 2 * H)

    r0p, w0p, w1p = _prep(xt, wave_w, wb, kw, kb)

    hid = _sc_gather(xt,
                     r0p.reshape(H * L * SP),
                     w0p.reshape(H * L * SP),
                     w1p.reshape(H * L * SP))
    out = _outproj(hid.reshape(H, L, D), out_w)
    return out[None]
